# baseline (speedup 1.0000x reference)
import functools

import jax
import jax.numpy as jnp
from jax import lax
from jax.experimental import pallas as pl
from jax.experimental.pallas import tpu as pltpu
from jax.experimental.pallas import tpu_sc as plsc

EMB_DIM = 1000
BATCH = 16384

NUM_CORES = 2
NUM_SUBCORES = 16
LANES = 16
NUM_WORKERS = NUM_CORES * NUM_SUBCORES

ROWS_PER_TILE = BATCH // NUM_WORKERS
CHUNK_ROWS = 64
CHUNK_WORDS = CHUNK_ROWS * EMB_DIM
NUM_CHUNKS = ROWS_PER_TILE // CHUNK_ROWS
GROUPS_PER_CHUNK = CHUNK_ROWS // LANES


def _scatter_chunk(buf, labels_v, chunk, value):
  lane_iota = lax.broadcasted_iota(jnp.int32, (LANES,), 0)
  vals = jnp.full((LANES,), value, jnp.float32)
  for g in range(GROUPS_PER_CHUNK):
    off = chunk * CHUNK_ROWS + g * LANES
    col_idx = labels_v[pl.ds(off, LANES)]
    flat_idx = (g * LANES + lane_iota) * EMB_DIM + col_idx
    plsc.store_scatter(buf, [flat_idx], vals)


def _one_hot_body(labels_hbm, out_hbm, labels_v, buf0, buf1, sem0, sem1):
  wid = lax.axis_index("s") * NUM_CORES + lax.axis_index("c")
  row_base = wid * ROWS_PER_TILE

  pltpu.sync_copy(labels_hbm.at[pl.ds(row_base, ROWS_PER_TILE)], labels_v)

  bufs = (buf0, buf1)
  sems = (sem0, sem1)
  zeros16 = jnp.zeros((LANES,), jnp.float32)

  def zero_buf(buf):
    def body(i, _):
      buf[pl.ds(i * LANES, LANES)] = zeros16
      return 0
    lax.fori_loop(0, CHUNK_WORDS // LANES, body, 0)

  copies = [None, None]
  for t in range(NUM_CHUNKS):
    slot = t % 2
    buf = bufs[slot]
    if t < 2:
      zero_buf(buf)
    else:
      copies[slot].wait()
      _scatter_chunk(buf, labels_v, t - 2, 0.0)
    _scatter_chunk(buf, labels_v, t, 1.0)
    copies[slot] = pltpu.async_copy(
        buf,
        out_hbm.at[pl.ds((row_base + t * CHUNK_ROWS) * EMB_DIM, CHUNK_WORDS)],
        sems[slot])
  copies[0].wait()
  copies[1].wait()


@jax.jit
def kernel(labels):
  mesh = plsc.VectorSubcoreMesh(
      core_axis_name="c", subcore_axis_name="s",
      num_cores=NUM_CORES, num_subcores=NUM_SUBCORES)
  flat = pl.kernel(
      _one_hot_body,
      out_type=jax.ShapeDtypeStruct((BATCH * EMB_DIM,), jnp.float32),
      mesh=mesh,
      scratch_types=[
          pltpu.VMEM((ROWS_PER_TILE,), jnp.int32),
          pltpu.VMEM((CHUNK_WORDS,), jnp.float32),
          pltpu.VMEM((CHUNK_WORDS,), jnp.float32),
          pltpu.SemaphoreType.DMA,
          pltpu.SemaphoreType.DMA,
      ],
      compiler_params=pltpu.CompilerParams(needs_layout_passes=False),
  )(labels.astype(jnp.int32))
  return flat.reshape(BATCH, EMB_DIM)


# speedup vs baseline: 1.0772x; 1.0772x over previous
import jax
import jax.numpy as jnp
from jax import lax
from jax.experimental import pallas as pl
from jax.experimental.pallas import tpu as pltpu
from jax.experimental.pallas import tpu_sc as plsc

EMB_DIM = 1000
BATCH = 16384

NUM_CORES = 2
NUM_SUBCORES = 16
LANES = 16
NUM_WORKERS = NUM_CORES * NUM_SUBCORES

ROWS_PER_TILE = BATCH // NUM_WORKERS
WORDS_PER_TILE = ROWS_PER_TILE * EMB_DIM
ZBUF_WORDS = 64000
NUM_ZDMA = WORDS_PER_TILE // ZBUF_WORDS
IDX_ROWS = 4
IDX_COLS = ROWS_PER_TILE // IDX_ROWS


def _one_hot_body(labels_hbm, out_hbm, labels_v, zbuf, idx_v, ones_v,
                  zsem, ssem):
  wid = lax.axis_index("s") * NUM_CORES + lax.axis_index("c")
  row_base = wid * ROWS_PER_TILE
  word_base = row_base * EMB_DIM

  pltpu.sync_copy(labels_hbm.at[pl.ds(row_base, ROWS_PER_TILE)], labels_v)

  zeros16 = jnp.zeros((LANES,), jnp.float32)
  def zfill(i, _):
    for u in range(4):
      zbuf[pl.ds((i * 4 + u) * LANES, LANES)] = zeros16
    return 0
  lax.fori_loop(0, ZBUF_WORDS // (4 * LANES), zfill, 0)

  zcopies = [
      pltpu.async_copy(
          zbuf, out_hbm.at[pl.ds(word_base + d * ZBUF_WORDS, ZBUF_WORDS)],
          zsem)
      for d in range(NUM_ZDMA)
  ]

  lane_iota = lax.broadcasted_iota(jnp.int32, (LANES,), 0)
  ones16 = jnp.full((LANES,), 1.0, jnp.float32)
  for g in range(ROWS_PER_TILE // LANES):
    lbl = labels_v[pl.ds(g * LANES, LANES)]
    flat = (word_base + g * LANES * EMB_DIM) + lane_iota * EMB_DIM + lbl
    idx_v[g // (IDX_COLS // LANES),
          pl.ds((g % (IDX_COLS // LANES)) * LANES, LANES)] = flat
    ones_v[pl.ds(g * LANES, LANES)] = ones16

  for c in zcopies:
    c.wait()

  scopies = [
      pltpu.async_copy(
          ones_v.at[pl.ds(j * IDX_COLS, IDX_COLS)],
          out_hbm.at[idx_v.at[j]],
          ssem)
      for j in range(IDX_ROWS)
  ]
  for c in scopies:
    c.wait()


@jax.jit
def kernel(labels):
  mesh = plsc.VectorSubcoreMesh(
      core_axis_name="c", subcore_axis_name="s",
      num_cores=NUM_CORES, num_subcores=NUM_SUBCORES)
  flat = pl.kernel(
      _one_hot_body,
      out_type=jax.ShapeDtypeStruct((BATCH * EMB_DIM,), jnp.float32),
      mesh=mesh,
      scratch_types=[
          pltpu.VMEM((ROWS_PER_TILE,), jnp.int32),
          pltpu.VMEM((ZBUF_WORDS,), jnp.float32),
          pltpu.VMEM((IDX_ROWS, IDX_COLS), jnp.int32),
          pltpu.VMEM((ROWS_PER_TILE,), jnp.float32),
          pltpu.SemaphoreType.DMA,
          pltpu.SemaphoreType.DMA,
      ],
      compiler_params=pltpu.CompilerParams(needs_layout_passes=False),
  )(labels.astype(jnp.int32))
  return flat.reshape(BATCH, EMB_DIM)


# speedup vs baseline: 1.7438x; 1.6188x over previous
import jax
import jax.numpy as jnp
from jax import lax
from jax.experimental import pallas as pl
from jax.experimental.pallas import tpu as pltpu
from jax.experimental.pallas import tpu_sc as plsc

EMB_DIM = 1000
BATCH = 16384

NUM_CORES = 2
NUM_SUBCORES = 16
LANES = 16
NUM_WORKERS = NUM_CORES * NUM_SUBCORES

ROWS_PER_TILE = BATCH // NUM_WORKERS
CHUNK_ROWS = 32
NUM_CHUNKS = ROWS_PER_TILE // CHUNK_ROWS
GROUPS_PER_CHUNK = CHUNK_ROWS // LANES


def _scatter_chunk(buf, labels_v, chunk, value):
  lane_iota = lax.broadcasted_iota(jnp.int32, (LANES,), 0)
  vals = jnp.full((LANES,), value, jnp.float32)
  for g in range(GROUPS_PER_CHUNK):
    off = chunk * CHUNK_ROWS + g * LANES
    col_idx = labels_v[pl.ds(off, LANES)]
    row_idx = g * LANES + lane_iota
    plsc.store_scatter(buf, [row_idx, col_idx], vals)


def _one_hot_body(labels_hbm, out_hbm, labels_v, buf0, buf1, sem0, sem1):
  wid = lax.axis_index("s") * NUM_CORES + lax.axis_index("c")
  row_base = wid * ROWS_PER_TILE

  pltpu.sync_copy(labels_hbm.at[pl.ds(row_base, ROWS_PER_TILE)], labels_v)

  bufs = (buf0, buf1)
  sems = (sem0, sem1)
  zeros16 = jnp.zeros((LANES,), jnp.float32)

  def zero_buf(buf):
    def body(r, _):
      def cbody(c, _):
        buf[r, pl.ds(c * LANES, LANES)] = zeros16
        return 0
      lax.fori_loop(0, EMB_DIM // LANES, cbody, 0)
      buf[r, pl.ds(EMB_DIM - LANES, LANES)] = zeros16
      return 0
    lax.fori_loop(0, CHUNK_ROWS, body, 0)

  copies = [None, None]
  for t in range(NUM_CHUNKS):
    slot = t % 2
    buf = bufs[slot]
    if t < 2:
      zero_buf(buf)
    else:
      copies[slot].wait()
      _scatter_chunk(buf, labels_v, t - 2, 0.0)
    _scatter_chunk(buf, labels_v, t, 1.0)
    copies[slot] = pltpu.async_copy(
        buf, out_hbm.at[pl.ds(row_base + t * CHUNK_ROWS, CHUNK_ROWS)],
        sems[slot])
  copies[0].wait()
  copies[1].wait()


@jax.jit
def kernel(labels):
  mesh = plsc.VectorSubcoreMesh(
      core_axis_name="c", subcore_axis_name="s",
      num_cores=NUM_CORES, num_subcores=NUM_SUBCORES)
  return pl.kernel(
      _one_hot_body,
      out_type=jax.ShapeDtypeStruct((BATCH, EMB_DIM), jnp.float32),
      mesh=mesh,
      scratch_types=[
          pltpu.VMEM((ROWS_PER_TILE,), jnp.int32),
          pltpu.VMEM((CHUNK_ROWS, EMB_DIM), jnp.float32),
          pltpu.VMEM((CHUNK_ROWS, EMB_DIM), jnp.float32),
          pltpu.SemaphoreType.DMA,
          pltpu.SemaphoreType.DMA,
      ],
      compiler_params=pltpu.CompilerParams(needs_layout_passes=False),
  )(labels.astype(jnp.int32))
